# Initial kernel scaffold; baseline (speedup 1.0000x reference)
import jax
import jax.numpy as jnp
from jax.experimental import pallas as pl


def kernel(x, Wg, fc1s, b1s, fc2s, b2s, ln_gamma, ln_beta):
    raise NotImplementedError("write your pallas kernel here")


# speedup vs baseline: 1.0235x; 1.0235x over previous
import functools

import jax
import jax.numpy as jnp
from jax.experimental import pallas as pl
from jax.experimental.pallas import tpu as pltpu


def _gelu_tanh(h):
    return 0.5 * h * (1.0 + jnp.tanh(jnp.sqrt(2.0 / jnp.pi) * (h + 0.044715 * h ** 3)))


def _mlp_body(w_ref, y_ref, fc1_ref, b1_ref, fc2_ref, b2_ref, gb_ref, o_ref):
    f = pl.program_id(2)
    nf = pl.num_programs(2)
    y = y_ref[0]
    mu = jnp.mean(y, axis=-1, keepdims=True)
    yc = y - mu
    var = jnp.mean(yc * yc, axis=-1, keepdims=True)
    yn = yc * jax.lax.rsqrt(var + 1e-5) * gb_ref[0, 0][None, :] + gb_ref[0, 1][None, :]
    h = jax.lax.dot_general(
        yn.astype(jnp.bfloat16), fc1_ref[0],
        dimension_numbers=(((1,), (1,)), ((), ())),
        preferred_element_type=jnp.float32,
    )
    h = _gelu_tanh(h + b1_ref[0])
    o = jax.lax.dot_general(
        h.astype(jnp.bfloat16), fc2_ref[0],
        dimension_numbers=(((1,), (1,)), ((), ())),
        preferred_element_type=jnp.float32,
    )

    @pl.when(f == 0)
    def _init():
        o_ref[0] = jnp.zeros_like(o_ref[0])

    o_ref[0] += o

    @pl.when(f == nf - 1)
    def _fin():
        o_ref[0] = (o_ref[0] + b2_ref[0, 0][None, :]) * w_ref[0, 0][:, None]


def _mlp_pallas(y, w, fc1, b1, fc2, b2, gb, *, tb, fb, interpret=False):
    E, k, d = y.shape
    dd = fc1.shape[1]
    nt = k // tb
    nf = dd // fb
    grid = (E, nt, nf)
    return pl.pallas_call(
        _mlp_body,
        grid=grid,
        in_specs=[
            pl.BlockSpec((1, 1, tb), lambda e, t, f: (e, 0, t)),
            pl.BlockSpec((1, tb, d), lambda e, t, f: (e, t, 0)),
            pl.BlockSpec((1, fb, d), lambda e, t, f: (e, f, 0)),
            pl.BlockSpec((1, 1, fb), lambda e, t, f: (e, 0, f)),
            pl.BlockSpec((1, d, fb), lambda e, t, f: (e, 0, f)),
            pl.BlockSpec((1, 1, d), lambda e, t, f: (e, 0, 0)),
            pl.BlockSpec((1, 2, d), lambda e, t, f: (0, 0, 0)),
        ],
        out_specs=pl.BlockSpec((1, tb, d), lambda e, t, f: (e, t, 0)),
        out_shape=jax.ShapeDtypeStruct((E, k, d), jnp.float32),
        compiler_params=pltpu.CompilerParams(
            dimension_semantics=("parallel", "parallel", "arbitrary"),
        ),
        interpret=interpret,
    )(w, y, fc1, b1, fc2, b2, gb)


def kernel(x, Wg, fc1s, b1s, fc2s, b2s, ln_gamma, ln_beta):
    og_shape = x.shape
    d = og_shape[-1]
    xf = x.reshape(-1, d)
    bs = xf.shape[0]
    E = Wg.shape[0]
    k = bs // E
    dd = fc1s.shape[1]

    scores = (jnp.tanh(xf @ Wg.T) + 1.0) / 2.0
    w_e, idx_e = jax.lax.top_k(scores.T, k)
    flat_idx = idx_e.reshape(-1)
    y = jnp.take(xf, flat_idx, axis=0).reshape(E, k, d)

    gb = jnp.stack([ln_gamma, ln_beta])[None]
    o = _mlp_pallas(
        y,
        w_e[:, None, :],
        fc1s.astype(jnp.bfloat16),
        b1s[:, None, :],
        fc2s.astype(jnp.bfloat16),
        b2s[:, None, :],
        gb,
        tb=min(512, k),
        fb=min(512, dd),
    )
    out = xf.at[flat_idx].add(o.reshape(-1, d))
    return out.reshape(og_shape)


# speedup vs baseline: 1.0857x; 1.0608x over previous
import functools

import jax
import jax.numpy as jnp
from jax.experimental import pallas as pl
from jax.experimental.pallas import tpu as pltpu


def _gelu_tanh(h):
    return 0.5 * h * (1.0 + jnp.tanh(jnp.sqrt(2.0 / jnp.pi) * (h + 0.044715 * h ** 3)))


def _mlp_body(w_ref, y_ref, fc1_ref, b1_ref, fc2_ref, b2_ref, gb_ref, o_ref, yn_ref):
    f = pl.program_id(2)
    nf = pl.num_programs(2)

    @pl.when(f == 0)
    def _ln():
        y = y_ref[0]
        mu = jnp.mean(y, axis=-1, keepdims=True)
        yc = y - mu
        var = jnp.mean(yc * yc, axis=-1, keepdims=True)
        yn = yc * jax.lax.rsqrt(var + 1e-5) * gb_ref[0, 0][None, :] + gb_ref[0, 1][None, :]
        yn_ref[...] = yn.astype(jnp.bfloat16)

    h = jnp.dot(yn_ref[...], fc1_ref[0], preferred_element_type=jnp.float32)
    h = _gelu_tanh(h + b1_ref[0])
    o = jnp.dot(h.astype(jnp.bfloat16), fc2_ref[0], preferred_element_type=jnp.float32)

    @pl.when(f == 0)
    def _init():
        o_ref[0] = jnp.zeros_like(o_ref[0])

    o_ref[0] += o

    @pl.when(f == nf - 1)
    def _fin():
        o_ref[0] = (o_ref[0] + b2_ref[0, 0][None, :]) * w_ref[0, 0][:, None]


def _mlp_pallas(y, w, fc1t, b1, fc2t, b2, gb, *, tb, fb, interpret=False):
    E, k, d = y.shape
    dd = fc1t.shape[2]
    nt = k // tb
    nf = dd // fb
    grid = (E, nt, nf)
    return pl.pallas_call(
        _mlp_body,
        grid=grid,
        in_specs=[
            pl.BlockSpec((1, 1, tb), lambda e, t, f: (e, 0, t)),
            pl.BlockSpec((1, tb, d), lambda e, t, f: (e, t, 0)),
            pl.BlockSpec((1, d, fb), lambda e, t, f: (e, 0, f)),
            pl.BlockSpec((1, 1, fb), lambda e, t, f: (e, 0, f)),
            pl.BlockSpec((1, fb, d), lambda e, t, f: (e, f, 0)),
            pl.BlockSpec((1, 1, d), lambda e, t, f: (e, 0, 0)),
            pl.BlockSpec((1, 2, d), lambda e, t, f: (0, 0, 0)),
        ],
        out_specs=pl.BlockSpec((1, tb, d), lambda e, t, f: (e, t, 0)),
        out_shape=jax.ShapeDtypeStruct((E, k, d), jnp.float32),
        scratch_shapes=[pltpu.VMEM((tb, d), jnp.bfloat16)],
        compiler_params=pltpu.CompilerParams(
            dimension_semantics=("arbitrary", "arbitrary", "arbitrary"),
            vmem_limit_bytes=128 * 1024 * 1024,
        ),
        interpret=interpret,
    )(w, y, fc1t, b1, fc2t, b2, gb)


def kernel(x, Wg, fc1s, b1s, fc2s, b2s, ln_gamma, ln_beta):
    og_shape = x.shape
    d = og_shape[-1]
    xf = x.reshape(-1, d)
    bs = xf.shape[0]
    E = Wg.shape[0]
    k = bs // E
    dd = fc1s.shape[1]

    scores = (jnp.tanh(xf @ Wg.T) + 1.0) / 2.0
    w_e, idx_e = jax.lax.top_k(scores.T, k)
    flat_idx = idx_e.reshape(-1)
    y = jnp.take(xf, flat_idx, axis=0).reshape(E, k, d)

    gb = jnp.stack([ln_gamma, ln_beta])[None]
    o = _mlp_pallas(
        y,
        w_e[:, None, :],
        jnp.swapaxes(fc1s, 1, 2).astype(jnp.bfloat16),
        b1s[:, None, :],
        jnp.swapaxes(fc2s, 1, 2).astype(jnp.bfloat16),
        b2s[:, None, :],
        gb,
        tb=min(2048, k),
        fb=min(512, dd),
    )
    out = xf.at[flat_idx].add(o.reshape(-1, d))
    return out.reshape(og_shape)


# speedup vs baseline: 1.1024x; 1.0154x over previous
import functools

import jax
import jax.numpy as jnp
from jax.experimental import pallas as pl
from jax.experimental.pallas import tpu as pltpu


def _gelu_tanh(h):
    return 0.5 * h * (1.0 + jnp.tanh(jnp.sqrt(2.0 / jnp.pi) * (h + 0.044715 * h ** 3)))


def _mlp_body(nf, w_ref, y_ref, fc1_ref, b1_ref, fc2_ref, b2_ref, gb_ref, o_ref,
              yn_ref, h_ref):
    p = pl.program_id(1)
    fb = fc1_ref.shape[2]

    @pl.when(p == 0)
    def _ln():
        y = y_ref[0]
        mu = jnp.mean(y, axis=-1, keepdims=True)
        yc = y - mu
        var = jnp.mean(yc * yc, axis=-1, keepdims=True)
        yn = yc * jax.lax.rsqrt(var + 1e-5) * gb_ref[0, 0][None, :] + gb_ref[0, 1][None, :]
        yn_ref[...] = yn.astype(jnp.bfloat16)

    @pl.when(p < nf)
    def _up():
        h = jnp.dot(yn_ref[...], fc1_ref[0], preferred_element_type=jnp.float32)
        h = _gelu_tanh(h + b1_ref[0])
        h_ref[:, pl.ds(pl.multiple_of(p * fb, fb), fb)] = h.astype(jnp.bfloat16)

    @pl.when(p >= nf)
    def _down():
        o = jnp.dot(h_ref[...], fc2_ref[0], preferred_element_type=jnp.float32)
        o_ref[0] = (o + b2_ref[0, 0][None, :]) * w_ref[0, 0][:, None]


def _mlp_pallas(y, w, fc1t, b1, fc2t, b2, gb, *, tb, fb, db, interpret=False):
    E, k, d = y.shape
    dd = fc1t.shape[2]
    nf = dd // fb
    nd = d // db
    grid = (E, nf + nd)

    def fmap(e, p):
        return (e, 0, jnp.minimum(p, nf - 1))

    def dmap(e, p):
        return (e, 0, jnp.maximum(p - nf, 0))

    return pl.pallas_call(
        functools.partial(_mlp_body, nf),
        grid=grid,
        in_specs=[
            pl.BlockSpec((1, 1, tb), lambda e, p: (e, 0, 0)),
            pl.BlockSpec((1, tb, d), lambda e, p: (e, 0, 0)),
            pl.BlockSpec((1, d, fb), fmap),
            pl.BlockSpec((1, 1, fb), fmap),
            pl.BlockSpec((1, dd, db), dmap),
            pl.BlockSpec((1, 1, db), dmap),
            pl.BlockSpec((1, 2, d), lambda e, p: (0, 0, 0)),
        ],
        out_specs=pl.BlockSpec((1, tb, db), dmap),
        out_shape=jax.ShapeDtypeStruct((E, k, d), jnp.float32),
        scratch_shapes=[
            pltpu.VMEM((tb, d), jnp.bfloat16),
            pltpu.VMEM((tb, dd), jnp.bfloat16),
        ],
        compiler_params=pltpu.CompilerParams(
            dimension_semantics=("arbitrary", "arbitrary"),
            vmem_limit_bytes=128 * 1024 * 1024,
        ),
        interpret=interpret,
    )(w, y, fc1t, b1, fc2t, b2, gb)


def kernel(x, Wg, fc1s, b1s, fc2s, b2s, ln_gamma, ln_beta):
    og_shape = x.shape
    d = og_shape[-1]
    xf = x.reshape(-1, d)
    bs = xf.shape[0]
    E = Wg.shape[0]
    k = bs // E
    dd = fc1s.shape[1]

    scores = (jnp.tanh(xf @ Wg.T) + 1.0) / 2.0
    w_e, idx_e = jax.lax.top_k(scores.T, k)
    flat_idx = idx_e.reshape(-1)
    y = jnp.take(xf, flat_idx, axis=0).reshape(E, k, d)

    gb = jnp.stack([ln_gamma, ln_beta])[None]
    o = _mlp_pallas(
        y,
        w_e[:, None, :],
        jnp.swapaxes(fc1s, 1, 2).astype(jnp.bfloat16),
        b1s[:, None, :],
        jnp.swapaxes(fc2s, 1, 2).astype(jnp.bfloat16),
        b2s[:, None, :],
        gb,
        tb=min(2048, k),
        fb=min(512, dd),
        db=min(256, d),
    )
    out = xf.at[flat_idx].add(o.reshape(-1, d))
    return out.reshape(og_shape)


# speedup vs baseline: 1.1834x; 1.0734x over previous
import functools

import jax
import jax.numpy as jnp
from jax import lax
from jax.experimental import pallas as pl
from jax.experimental.pallas import tpu as pltpu
from jax.experimental.pallas import tpu_sc as plsc


def _gelu_tanh(h):
    return 0.5 * h * (1.0 + jnp.tanh(jnp.sqrt(2.0 / jnp.pi) * (h + 0.044715 * h ** 3)))


def _mlp_body(nf, w_ref, y_ref, fc1_ref, b1_ref, fc2_ref, b2_ref, gb_ref, o_ref,
              yn_ref, h_ref):
    p = pl.program_id(1)
    fb = fc1_ref.shape[2]

    @pl.when(p == 0)
    def _ln():
        y = y_ref[0]
        mu = jnp.mean(y, axis=-1, keepdims=True)
        yc = y - mu
        var = jnp.mean(yc * yc, axis=-1, keepdims=True)
        yn = yc * jax.lax.rsqrt(var + 1e-5) * gb_ref[0, 0][None, :] + gb_ref[0, 1][None, :]
        yn_ref[...] = yn.astype(jnp.bfloat16)

    @pl.when(p < nf)
    def _up():
        h = jnp.dot(yn_ref[...], fc1_ref[0], preferred_element_type=jnp.float32)
        h = _gelu_tanh(h + b1_ref[0])
        h_ref[:, pl.ds(pl.multiple_of(p * fb, fb), fb)] = h.astype(jnp.bfloat16)

    @pl.when(p >= nf)
    def _down():
        o = jnp.dot(h_ref[...], fc2_ref[0], preferred_element_type=jnp.float32)
        o_ref[0] = (o + b2_ref[0, 0][None, :]) * w_ref[0, 0][:, None]


def _mlp_pallas(y, w, fc1t, b1, fc2t, b2, gb, *, tb, fb, db, interpret=False):
    E, k, d = y.shape
    dd = fc1t.shape[2]
    nf = dd // fb
    nd = d // db
    grid = (E, nf + nd)

    def fmap(e, p):
        return (e, 0, jnp.minimum(p, nf - 1))

    def dmap(e, p):
        return (e, 0, jnp.maximum(p - nf, 0))

    return pl.pallas_call(
        functools.partial(_mlp_body, nf),
        grid=grid,
        in_specs=[
            pl.BlockSpec((1, 1, tb), lambda e, p: (e, 0, 0)),
            pl.BlockSpec((1, tb, d), lambda e, p: (e, 0, 0)),
            pl.BlockSpec((1, d, fb), fmap),
            pl.BlockSpec((1, 1, fb), fmap),
            pl.BlockSpec((1, dd, db), dmap),
            pl.BlockSpec((1, 1, db), dmap),
            pl.BlockSpec((1, 2, d), lambda e, p: (0, 0, 0)),
        ],
        out_specs=pl.BlockSpec((1, tb, db), dmap),
        out_shape=jax.ShapeDtypeStruct((E, k, d), jnp.float32),
        scratch_shapes=[
            pltpu.VMEM((tb, d), jnp.bfloat16),
            pltpu.VMEM((tb, dd), jnp.bfloat16),
        ],
        compiler_params=pltpu.CompilerParams(
            dimension_semantics=("arbitrary", "arbitrary"),
            vmem_limit_bytes=128 * 1024 * 1024,
        ),
        interpret=interpret,
    )(w, y, fc1t, b1, fc2t, b2, gb)


def _combine_pallas(xf, o_flat, idx3, *, cc=128):
    bs, d = xf.shape
    nc, ns = 2, 16
    nci = d // cc // nc
    nh = 4
    half = bs // nh
    spt = half // ns
    rpt = bs // ns
    nj = rpt // 128

    mesh = plsc.VectorSubcoreMesh(core_axis_name="c", subcore_axis_name="s")

    @functools.partial(
        pl.kernel,
        mesh=mesh,
        out_type=jax.ShapeDtypeStruct((bs, d), jnp.float32),
        scratch_types=[
            pltpu.VMEM((512, cc), jnp.float32),
            pltpu.VMEM((nj, 128), jnp.int32),
            pltpu.VMEM((4, nj, 128), jnp.int32),
            pltpu.VMEM_SHARED((half + 128, cc), jnp.float32),
        ],
    )
    def k(xf_hbm, o_hbm, idx_hbm, out_hbm, buf, idxb, idxl, spm):
        c = lax.axis_index("c")
        s = lax.axis_index("s")
        pltpu.sync_copy(idx_hbm.at[s], idxb)
        for h in range(nh):
            for j in range(nj):
                for q in range(128 // 16):
                    v = idxb[j, pl.ds(q * 16, 16)]
                    lv = v - h * half
                    ok = (lv >= 0) & (lv < half)
                    idxl[h, j, pl.ds(q * 16, 16)] = jnp.where(
                        ok, lv, half + (v & 127))
        for ci in range(nci):
            col0 = (c * nci + ci) * cc
            for h in range(nh):
                r0 = h * half + s * spt
                pltpu.sync_copy(xf_hbm.at[pl.ds(r0, spt), pl.ds(col0, cc)],
                                buf.at[pl.ds(0, spt)])
                pltpu.sync_copy(buf.at[pl.ds(0, spt)], spm.at[pl.ds(s * spt, spt)])
                plsc.subcore_barrier()
                for hb in range(rpt // 512):
                    pltpu.sync_copy(
                        o_hbm.at[pl.ds(s * rpt + hb * 512, 512), pl.ds(col0, cc)],
                        buf)
                    for j in range(4):
                        pltpu.sync_copy(buf.at[pl.ds(j * 128, 128)],
                                        spm.at[idxl.at[h, hb * 4 + j]], add=True)
                plsc.subcore_barrier()
                pltpu.sync_copy(spm.at[pl.ds(s * spt, spt)], buf.at[pl.ds(0, spt)])
                pltpu.sync_copy(buf.at[pl.ds(0, spt)],
                                out_hbm.at[pl.ds(r0, spt), pl.ds(col0, cc)])
                plsc.subcore_barrier()

    return k(xf, o_flat, idx3)


def kernel(x, Wg, fc1s, b1s, fc2s, b2s, ln_gamma, ln_beta):
    og_shape = x.shape
    d = og_shape[-1]
    xf = x.reshape(-1, d)
    bs = xf.shape[0]
    E = Wg.shape[0]
    k = bs // E
    dd = fc1s.shape[1]

    scores = (jnp.tanh(xf @ Wg.T) + 1.0) / 2.0
    w_e, idx_e = jax.lax.top_k(scores.T, k)
    flat_idx = idx_e.reshape(-1)
    y = jnp.take(xf, flat_idx, axis=0).reshape(E, k, d)

    gb = jnp.stack([ln_gamma, ln_beta])[None]
    o = _mlp_pallas(
        y,
        w_e[:, None, :],
        jnp.swapaxes(fc1s, 1, 2).astype(jnp.bfloat16),
        b1s[:, None, :],
        jnp.swapaxes(fc2s, 1, 2).astype(jnp.bfloat16),
        b2s[:, None, :],
        gb,
        tb=min(2048, k),
        fb=min(512, dd),
        db=min(512, d),
    )
    if bs == 16384 and d % 128 == 0:
        idx3 = flat_idx.astype(jnp.int32).reshape(16, bs // 16 // 128, 128)
        out = _combine_pallas(xf, o.reshape(-1, d), idx3)
    else:
        out = xf.at[flat_idx].add(o.reshape(-1, d))
    return out.reshape(og_shape)


# speedup vs baseline: 1.2280x; 1.0377x over previous
import functools

import jax
import jax.numpy as jnp
from jax import lax
from jax.experimental import pallas as pl
from jax.experimental.pallas import tpu as pltpu
from jax.experimental.pallas import tpu_sc as plsc


def _gelu_tanh(h):
    return 0.5 * h * (1.0 + jnp.tanh(jnp.sqrt(2.0 / jnp.pi) * (h + 0.044715 * h ** 3)))


def _mlp_body(nf, w_ref, y_ref, fc1_ref, b1_ref, fc2_ref, b2_ref, gb_ref, o_ref,
              yn_ref, h_ref):
    p = pl.program_id(1)
    fb = fc1_ref.shape[2]

    @pl.when(p == 0)
    def _ln():
        y = y_ref[0]
        mu = jnp.mean(y, axis=-1, keepdims=True)
        yc = y - mu
        var = jnp.mean(yc * yc, axis=-1, keepdims=True)
        yn = yc * jax.lax.rsqrt(var + 1e-5) * gb_ref[0, 0][None, :] + gb_ref[0, 1][None, :]
        yn_ref[...] = yn.astype(jnp.bfloat16)

    @pl.when(p < nf)
    def _up():
        h = jnp.dot(yn_ref[...], fc1_ref[0], preferred_element_type=jnp.float32)
        h = _gelu_tanh(h + b1_ref[0])
        h_ref[:, pl.ds(pl.multiple_of(p * fb, fb), fb)] = h.astype(jnp.bfloat16)

    @pl.when(p >= nf)
    def _down():
        o = jnp.dot(h_ref[...], fc2_ref[0], preferred_element_type=jnp.float32)
        o_ref[0] = (o + b2_ref[0, 0][None, :]) * w_ref[0, 0][:, None]


def _mlp_pallas(y, w, fc1t, b1, fc2t, b2, gb, *, tb, fb, db, interpret=False):
    E, k, d = y.shape
    dd = fc1t.shape[2]
    nf = dd // fb
    nd = d // db
    grid = (E, nf + nd)

    def fmap(e, p):
        return (e, 0, jnp.minimum(p, nf - 1))

    def dmap(e, p):
        return (e, 0, jnp.maximum(p - nf, 0))

    return pl.pallas_call(
        functools.partial(_mlp_body, nf),
        grid=grid,
        in_specs=[
            pl.BlockSpec((1, 1, tb), lambda e, p: (e, 0, 0)),
            pl.BlockSpec((1, tb, d), lambda e, p: (e, 0, 0)),
            pl.BlockSpec((1, d, fb), fmap),
            pl.BlockSpec((1, 1, fb), fmap),
            pl.BlockSpec((1, dd, db), dmap),
            pl.BlockSpec((1, 1, db), dmap),
            pl.BlockSpec((1, 2, d), lambda e, p: (0, 0, 0)),
        ],
        out_specs=pl.BlockSpec((1, tb, db), dmap),
        out_shape=jax.ShapeDtypeStruct((E, k, d), jnp.float32),
        scratch_shapes=[
            pltpu.VMEM((tb, d), jnp.bfloat16),
            pltpu.VMEM((tb, dd), jnp.bfloat16),
        ],
        compiler_params=pltpu.CompilerParams(
            dimension_semantics=("arbitrary", "arbitrary"),
            vmem_limit_bytes=128 * 1024 * 1024,
        ),
        interpret=interpret,
    )(w, y, fc1t, b1, fc2t, b2, gb)


def _gather_pallas(xf, idx3, *, bt=32):
    bs, d = xf.shape
    nc, ns = 2, 16
    nw = nc * ns
    spt = bs // nw
    nbt = spt // bt

    mesh = plsc.VectorSubcoreMesh(core_axis_name="c", subcore_axis_name="s")

    @functools.partial(
        pl.kernel,
        mesh=mesh,
        out_type=jax.ShapeDtypeStruct((bs, d), jnp.float32),
        scratch_types=[
            pltpu.VMEM((nbt, bt), jnp.int32),
            pltpu.VMEM((2, bt, d), jnp.float32),
            pltpu.SemaphoreType.DMA,
        ],
    )
    def k(xf_hbm, idx_hbm, y_hbm, idxb, buf, gsem):
        c = lax.axis_index("c")
        s = lax.axis_index("s")
        wid = s * nc + c
        base = wid * spt
        pltpu.sync_copy(idx_hbm.at[wid], idxb)
        hs = {}
        hs[0] = pltpu.async_copy(xf_hbm.at[idxb.at[0]], buf.at[0], gsem)
        for b in range(nbt):
            cur = b & 1
            if b + 1 < nbt:
                hs[b + 1] = pltpu.async_copy(
                    xf_hbm.at[idxb.at[b + 1]], buf.at[1 - cur], gsem)
            hs[b].wait()
            pltpu.sync_copy(buf.at[cur], y_hbm.at[pl.ds(base + b * bt, bt)])

    return k(xf, idx3)


def _combine_pallas(xf, o_flat, idx3, *, cc=128):
    bs, d = xf.shape
    nc, ns = 2, 16
    nci = d // cc // nc
    nh = 4
    half = bs // nh
    spt = half // ns
    rpt = bs // ns
    nj = rpt // 128

    mesh = plsc.VectorSubcoreMesh(core_axis_name="c", subcore_axis_name="s")

    @functools.partial(
        pl.kernel,
        mesh=mesh,
        out_type=jax.ShapeDtypeStruct((bs, d), jnp.float32),
        scratch_types=[
            pltpu.VMEM((512, cc), jnp.float32),
            pltpu.VMEM((nj, 128), jnp.int32),
            pltpu.VMEM((4, nj, 128), jnp.int32),
            pltpu.VMEM_SHARED((half + 128, cc), jnp.float32),
        ],
    )
    def k(xf_hbm, o_hbm, idx_hbm, out_hbm, buf, idxb, idxl, spm):
        c = lax.axis_index("c")
        s = lax.axis_index("s")
        pltpu.sync_copy(idx_hbm.at[s], idxb)
        for h in range(nh):
            for j in range(nj):
                for q in range(128 // 16):
                    v = idxb[j, pl.ds(q * 16, 16)]
                    lv = v - h * half
                    ok = (lv >= 0) & (lv < half)
                    idxl[h, j, pl.ds(q * 16, 16)] = jnp.where(
                        ok, lv, half + (v & 127))
        for ci in range(nci):
            col0 = (c * nci + ci) * cc
            for h in range(nh):
                r0 = h * half + s * spt
                pltpu.sync_copy(xf_hbm.at[pl.ds(r0, spt), pl.ds(col0, cc)],
                                buf.at[pl.ds(0, spt)])
                pltpu.sync_copy(buf.at[pl.ds(0, spt)], spm.at[pl.ds(s * spt, spt)])
                plsc.subcore_barrier()
                for hb in range(rpt // 512):
                    pltpu.sync_copy(
                        o_hbm.at[pl.ds(s * rpt + hb * 512, 512), pl.ds(col0, cc)],
                        buf)
                    for j in range(4):
                        pltpu.sync_copy(buf.at[pl.ds(j * 128, 128)],
                                        spm.at[idxl.at[h, hb * 4 + j]], add=True)
                plsc.subcore_barrier()
                pltpu.sync_copy(spm.at[pl.ds(s * spt, spt)], buf.at[pl.ds(0, spt)])
                pltpu.sync_copy(buf.at[pl.ds(0, spt)],
                                out_hbm.at[pl.ds(r0, spt), pl.ds(col0, cc)])
                plsc.subcore_barrier()

    return k(xf, o_flat, idx3)


def kernel(x, Wg, fc1s, b1s, fc2s, b2s, ln_gamma, ln_beta):
    og_shape = x.shape
    d = og_shape[-1]
    xf = x.reshape(-1, d)
    bs = xf.shape[0]
    E = Wg.shape[0]
    k = bs // E
    dd = fc1s.shape[1]

    scores = (jnp.tanh(xf @ Wg.T) + 1.0) / 2.0
    w_e, idx_e = jax.lax.top_k(scores.T, k)
    flat_idx = idx_e.reshape(-1)
    sc_ok = bs == 16384 and d % 128 == 0
    if sc_ok:
        idx3g = flat_idx.astype(jnp.int32).reshape(32, bs // 32 // 32, 32)
        y = _gather_pallas(xf, idx3g).reshape(E, k, d)
    else:
        y = jnp.take(xf, flat_idx, axis=0).reshape(E, k, d)

    gb = jnp.stack([ln_gamma, ln_beta])[None]
    o = _mlp_pallas(
        y,
        w_e[:, None, :],
        jnp.swapaxes(fc1s, 1, 2).astype(jnp.bfloat16),
        b1s[:, None, :],
        jnp.swapaxes(fc2s, 1, 2).astype(jnp.bfloat16),
        b2s[:, None, :],
        gb,
        tb=min(2048, k),
        fb=min(512, dd),
        db=min(512, d),
    )
    if sc_ok:
        idx3 = flat_idx.astype(jnp.int32).reshape(16, bs // 16 // 128, 128)
        out = _combine_pallas(xf, o.reshape(-1, d), idx3)
    else:
        out = xf.at[flat_idx].add(o.reshape(-1, d))
    return out.reshape(og_shape)
